# baseline (speedup 1.0000x reference)
import jax
import jax.numpy as jnp
from jax.experimental import pallas as pl
from jax.experimental.pallas import tpu as pltpu

_ALPHA = 4
_FAST_CHUNKS = 8


def kernel(frames):
    C, T, H, W = frames.shape
    n_slow = T // _ALPHA
    idx = jnp.linspace(0.0, T - 1, n_slow).astype(jnp.int32)

    chunk = T // _FAST_CHUNKS

    def body(idx_ref, in_ref, slow_ref, fast_ref, sem_fast, sem_slow):
        fast_dmas = [
            pltpu.make_async_copy(
                in_ref.at[:, pl.ds(k * chunk, chunk)],
                fast_ref.at[:, pl.ds(k * chunk, chunk)],
                sem_fast,
            )
            for k in range(_FAST_CHUNKS)
        ]
        slow_dmas = [
            pltpu.make_async_copy(
                in_ref.at[:, pl.ds(idx_ref[j], 1)],
                slow_ref.at[:, pl.ds(j, 1)],
                sem_slow,
            )
            for j in range(n_slow)
        ]
        for dma in fast_dmas + slow_dmas:
            dma.start()
        for dma in fast_dmas + slow_dmas:
            dma.wait()

    slow, fast = pl.pallas_call(
        body,
        grid=(),
        in_specs=[
            pl.BlockSpec(memory_space=pltpu.SMEM),
            pl.BlockSpec(memory_space=pl.ANY),
        ],
        out_specs=[
            pl.BlockSpec(memory_space=pl.ANY),
            pl.BlockSpec(memory_space=pl.ANY),
        ],
        out_shape=(
            jax.ShapeDtypeStruct((C, n_slow, H, W), frames.dtype),
            jax.ShapeDtypeStruct((C, T, H, W), frames.dtype),
        ),
        scratch_shapes=[pltpu.SemaphoreType.DMA, pltpu.SemaphoreType.DMA],
    )(idx, frames)
    return (slow, fast)


# speedup vs baseline: 17.9748x; 17.9748x over previous
import jax
import jax.numpy as jnp
from jax.experimental import pallas as pl
from jax.experimental.pallas import tpu as pltpu

_ALPHA = 4
_H_SPLIT = 2


def _pack_body(jmap_ref, in_ref, slow_ref, fast_ref):
    fast_ref[...] = in_ref[...]
    slow_ref[...] = in_ref[...]


def kernel(frames):
    C, T, H, W = frames.shape
    n_slow = T // _ALPHA
    hc = H // _H_SPLIT
    idx = jnp.linspace(0.0, T - 1, n_slow).astype(jnp.int32)
    jmap = jnp.searchsorted(idx, jnp.arange(T, dtype=jnp.int32)).astype(jnp.int32)

    grid_spec = pltpu.PrefetchScalarGridSpec(
        num_scalar_prefetch=1,
        grid=(_H_SPLIT, T),
        in_specs=[
            pl.BlockSpec((C, 1, hc, W), lambda h, t, jm: (0, t, h, 0)),
        ],
        out_specs=[
            pl.BlockSpec((C, 1, hc, W), lambda h, t, jm: (0, jm[t], h, 0)),
            pl.BlockSpec((C, 1, hc, W), lambda h, t, jm: (0, t, h, 0)),
        ],
    )
    slow, fast = pl.pallas_call(
        _pack_body,
        grid_spec=grid_spec,
        out_shape=(
            jax.ShapeDtypeStruct((C, n_slow, H, W), frames.dtype),
            jax.ShapeDtypeStruct((C, T, H, W), frames.dtype),
        ),
        compiler_params=pltpu.CompilerParams(
            dimension_semantics=("parallel", "arbitrary"),
        ),
    )(jmap, frames)
    return (slow, fast)


# speedup vs baseline: 49.3375x; 2.7448x over previous
import jax
import jax.numpy as jnp
from jax.experimental import pallas as pl
from jax.experimental.pallas import tpu as pltpu

_ALPHA = 4
_FPB = 8


def kernel(frames):
    C, T, H, W = frames.shape
    n_slow = T // _ALPHA
    spb = _FPB // _ALPHA
    idx = jnp.linspace(0.0, T - 1, n_slow).astype(jnp.int32)
    off = idx - _ALPHA * jnp.arange(n_slow, dtype=jnp.int32)

    def body(off_ref, in_ref, slow_ref, fast_ref):
        fast_ref[...] = in_ref[...]
        g = pl.program_id(0)
        for s in range(spb):
            o = off_ref[g * spb + s] + s * _ALPHA
            slow_ref[:, pl.ds(s, 1)] = in_ref[:, pl.ds(o, 1)]

    grid_spec = pltpu.PrefetchScalarGridSpec(
        num_scalar_prefetch=1,
        grid=(T // _FPB,),
        in_specs=[
            pl.BlockSpec((C, _FPB, H, W), lambda g, off_r: (0, g, 0, 0)),
        ],
        out_specs=[
            pl.BlockSpec((C, spb, H, W), lambda g, off_r: (0, g, 0, 0)),
            pl.BlockSpec((C, _FPB, H, W), lambda g, off_r: (0, g, 0, 0)),
        ],
    )
    slow, fast = pl.pallas_call(
        body,
        grid_spec=grid_spec,
        out_shape=(
            jax.ShapeDtypeStruct((C, n_slow, H, W), frames.dtype),
            jax.ShapeDtypeStruct((C, T, H, W), frames.dtype),
        ),
        compiler_params=pltpu.CompilerParams(
            dimension_semantics=("arbitrary",),
        ),
    )(off, frames)
    return (slow, fast)


# speedup vs baseline: 52.2133x; 1.0583x over previous
import jax
import jax.numpy as jnp
from jax.experimental import pallas as pl
from jax.experimental.pallas import tpu as pltpu

_ALPHA = 4
_FPB = 16


def kernel(frames):
    C, T, H, W = frames.shape
    n_slow = T // _ALPHA
    spb = _FPB // _ALPHA
    idx = jnp.linspace(0.0, T - 1, n_slow).astype(jnp.int32)
    off = idx - _ALPHA * jnp.arange(n_slow, dtype=jnp.int32)

    def body(off_ref, in_ref, slow_ref, fast_ref):
        fast_ref[...] = in_ref[...]
        g = pl.program_id(0)
        for s in range(spb):
            o = off_ref[g * spb + s] + s * _ALPHA
            slow_ref[:, pl.ds(s, 1)] = in_ref[:, pl.ds(o, 1)]

    grid_spec = pltpu.PrefetchScalarGridSpec(
        num_scalar_prefetch=1,
        grid=(T // _FPB,),
        in_specs=[
            pl.BlockSpec((C, _FPB, H, W), lambda g, off_r: (0, g, 0, 0)),
        ],
        out_specs=[
            pl.BlockSpec((C, spb, H, W), lambda g, off_r: (0, g, 0, 0)),
            pl.BlockSpec((C, _FPB, H, W), lambda g, off_r: (0, g, 0, 0)),
        ],
    )
    slow, fast = pl.pallas_call(
        body,
        grid_spec=grid_spec,
        out_shape=(
            jax.ShapeDtypeStruct((C, n_slow, H, W), frames.dtype),
            jax.ShapeDtypeStruct((C, T, H, W), frames.dtype),
        ),
        compiler_params=pltpu.CompilerParams(
            dimension_semantics=("arbitrary",),
        ),
    )(off, frames)
    return (slow, fast)
